# baseline (speedup 1.0000x reference)
import functools

import jax
import jax.numpy as jnp
from jax import lax
from jax.experimental import pallas as pl
from jax.experimental.pallas import tpu as pltpu
from jax.experimental.pallas import tpu_sc as plsc

_DIM = 64
_BATCH = 16384
_NNEG = 20
_NPAIR = _NNEG + 1
_NW = 32
_CH = 128
_NB = 4

_U_PER_W = _BATCH // _NW
_C_PER_W = _NPAIR * _BATCH // _NW


def _gather_loop(table, idx_hbm, out_hbm, base, ngroups, idx_bufs, row_bufs,
                 isems, osems):
    def group(g, carry):
        gathers = []
        for b in range(_NB):
            off = base + (g * _NB + b) * _CH
            pltpu.sync_copy(idx_hbm.at[pl.ds(off, _CH)], idx_bufs[b])
            gathers.append(pltpu.async_copy(table.at[idx_bufs[b]], row_bufs[b],
                                            isems[b]))
        outs = []
        for b in range(_NB):
            gathers[b].wait()
            off = base + (g * _NB + b) * _CH
            outs.append(pltpu.async_copy(row_bufs[b],
                                         out_hbm.at[pl.ds(off, _CH)], osems[b]))
        for b in range(_NB):
            outs[b].wait()
        return carry

    lax.fori_loop(0, ngroups, group, 0)


def _sc_gather_body(w_word, w_ctx, idx_u, idx_c, out_u, out_c, *scratch):
    idx_bufs = scratch[0:_NB]
    row_bufs = scratch[_NB:2 * _NB]
    isems = scratch[2 * _NB:3 * _NB]
    osems = scratch[3 * _NB:4 * _NB]
    wid = lax.axis_index("s") * 2 + lax.axis_index("c")
    _gather_loop(w_word, idx_u, out_u, wid * _U_PER_W, _U_PER_W // (_NB * _CH),
                 idx_bufs, row_bufs, isems, osems)
    _gather_loop(w_ctx, idx_c, out_c, wid * _C_PER_W, _C_PER_W // (_NB * _CH),
                 idx_bufs, row_bufs, isems, osems)


@functools.cache
def _sc_gather():
    return pl.kernel(
        _sc_gather_body,
        out_type=(
            jax.ShapeDtypeStruct((_BATCH, 2 * _DIM), jnp.float32),
            jax.ShapeDtypeStruct((_NPAIR * _BATCH, 2 * _DIM), jnp.float32),
        ),
        mesh=plsc.VectorSubcoreMesh(core_axis_name="c", subcore_axis_name="s"),
        scratch_types=(
            [pltpu.VMEM((_CH,), jnp.int32) for _ in range(_NB)]
            + [pltpu.VMEM((_CH, 2 * _DIM), jnp.float32) for _ in range(_NB)]
            + [pltpu.SemaphoreType.DMA for _ in range(2 * _NB)]
        ),
    )


_POLY = (-0.3665129217103557, 0.7213474868739994, -0.07983418559330518,
         -0.004969245784256061, 0.0023726909102893498, 0.0002506231232823097,
         -0.00013466852747399117, -1.3739849051717102e-05, 7.83856401653793e-06)


def _f_poly(t):
    acc = jnp.full_like(t, _POLY[-1])
    for c in _POLY[-2::-1]:
        acc = acc * t + c
    return acc


def _tc_body(u_ref, c_ref, s_ref, vols_ref, ints_ref, tv_ref, zZu_ref):
    j = pl.program_id(1)
    S = s_ref[...]

    @pl.when(j == 0)
    def _():
        u = u_ref[...]
        su = jax.nn.sigmoid(u)
        zu0 = su[:, :_DIM]
        Zu0 = zu0 + su[:, _DIM:] * (1.0 - zu0)
        zZu_ref[...] = jnp.concatenate([zu0, Zu0], axis=1)
        tu = Zu0 - zu0
        fu = _f_poly(jnp.concatenate([tu, tu], axis=1))
        tv_ref[0, :, 0] = jnp.dot(fu, S,
                                  preferred_element_type=jnp.float32)[:, 0]

    zu = zZu_ref[:, :_DIM]
    Zu = zZu_ref[:, _DIM:]
    c = c_ref[...]
    sc = jax.nn.sigmoid(c)
    zc = sc[:, :_DIM]
    Zc = zc + sc[:, _DIM:] * (1.0 - zc)
    t128 = jnp.concatenate(
        [Zc - zc, jnp.minimum(Zc, Zu) - jnp.maximum(zc, zu)], axis=1)
    r = jnp.dot(_f_poly(t128), S, preferred_element_type=jnp.float32)
    vols_ref[0, :, 0] = r[:, 0]
    ints_ref[0, :, 0] = r[:, 1]


def _tc_compute(u_rows, ctx_rows, bb=512):
    nb = _BATCH // bb
    ssum = jnp.zeros((2 * _DIM, 128), jnp.float32)
    ssum = ssum.at[:_DIM, 0].set(1.0).at[_DIM:, 1].set(1.0)
    return pl.pallas_call(
        _tc_body,
        grid=(nb, _NPAIR),
        in_specs=[
            pl.BlockSpec((bb, 2 * _DIM), lambda i, j: (i, 0)),
            pl.BlockSpec((bb, 2 * _DIM), lambda i, j, nb=nb: (j * nb + i, 0)),
            pl.BlockSpec((2 * _DIM, 128), lambda i, j: (0, 0)),
        ],
        out_specs=[
            pl.BlockSpec((1, bb, 1), lambda i, j: (j, i, 0)),
            pl.BlockSpec((1, bb, 1), lambda i, j: (j, i, 0)),
            pl.BlockSpec((1, bb, 1), lambda i, j: (0, i, 0)),
        ],
        out_shape=[
            jax.ShapeDtypeStruct((_NPAIR, _BATCH, 1), jnp.float32),
            jax.ShapeDtypeStruct((_NPAIR, _BATCH, 1), jnp.float32),
            jax.ShapeDtypeStruct((1, _BATCH, 1), jnp.float32),
        ],
        scratch_shapes=[pltpu.VMEM((bb, 2 * _DIM), jnp.float32)],
    )(u_rows, ctx_rows, ssum)


def kernel(pos_u, pos_w, neg_w, W_word, W_ctx):
    pos_u = pos_u.astype(jnp.int32)
    idx_ctx = jnp.concatenate(
        [neg_w.astype(jnp.int32).T.reshape(-1), pos_w.astype(jnp.int32)])
    u_rows, ctx_rows = _sc_gather()(W_word, W_ctx, pos_u, idx_ctx)
    vols, ints, tv = _tc_compute(u_rows, ctx_rows)
    vols, ints, tv = vols[..., 0], ints[..., 0], tv[..., 0]
    return (tv[0], vols[_NNEG], vols[:_NNEG].T, ints[_NNEG], ints[:_NNEG].T)


# speedup vs baseline: 1.6664x; 1.6664x over previous
import functools

import jax
import jax.numpy as jnp
from jax import lax
from jax.experimental import pallas as pl
from jax.experimental.pallas import tpu as pltpu
from jax.experimental.pallas import tpu_sc as plsc

_DIM = 64
_BATCH = 16384
_NNEG = 20
_NPAIR = _NNEG + 1
_NW = 32
_CH = 128
_NB = 4

_U_PER_W = _BATCH // _NW
_C_PER_W = _NPAIR * _BATCH // _NW


def _gather_loop(table, idx_hbm, out_hbm, base, ngroups, idx_bufs, row_bufs,
                 isems, osems):
    def group(g, carry):
        gathers = []
        for b in range(_NB):
            off = base + (g * _NB + b) * _CH
            pltpu.sync_copy(idx_hbm.at[pl.ds(off, _CH)], idx_bufs[b])
            gathers.append(pltpu.async_copy(table.at[idx_bufs[b]], row_bufs[b],
                                            isems[b]))
        outs = []
        for b in range(_NB):
            gathers[b].wait()
            off = base + (g * _NB + b) * _CH
            outs.append(pltpu.async_copy(row_bufs[b],
                                         out_hbm.at[pl.ds(off, _CH)], osems[b]))
        for b in range(_NB):
            outs[b].wait()
        return carry

    lax.fori_loop(0, ngroups, group, 0)


def _sc_gather_body(w_word, w_ctx, idx_u, idx_c, out_u, out_c, *scratch):
    idx_bufs = scratch[0:_NB]
    row_bufs = scratch[_NB:2 * _NB]
    isems = scratch[2 * _NB:3 * _NB]
    osems = scratch[3 * _NB:4 * _NB]
    wid = lax.axis_index("s") * 2 + lax.axis_index("c")
    _gather_loop(w_word, idx_u, out_u, wid * _U_PER_W, _U_PER_W // (_NB * _CH),
                 idx_bufs, row_bufs, isems, osems)
    _gather_loop(w_ctx, idx_c, out_c, wid * _C_PER_W, _C_PER_W // (_NB * _CH),
                 idx_bufs, row_bufs, isems, osems)


@functools.cache
def _sc_gather():
    return pl.kernel(
        _sc_gather_body,
        out_type=(
            jax.ShapeDtypeStruct((_BATCH, 2 * _DIM), jnp.float32),
            jax.ShapeDtypeStruct((_NPAIR * _BATCH, 2 * _DIM), jnp.float32),
        ),
        mesh=plsc.VectorSubcoreMesh(core_axis_name="c", subcore_axis_name="s"),
        scratch_types=(
            [pltpu.VMEM((_CH,), jnp.int32) for _ in range(_NB)]
            + [pltpu.VMEM((_CH, 2 * _DIM), jnp.float32) for _ in range(_NB)]
            + [pltpu.SemaphoreType.DMA for _ in range(2 * _NB)]
        ),
    )


_POLY = (-0.3665129829491377, 0.7213459840780112, -0.07983222595229246,
         -0.004957223416339075, 0.0023628927052507724, 0.00022657838744351012,
         -0.00011899139943125192)


def _f_poly(t):
    acc = jnp.full_like(t, _POLY[-1])
    for c in _POLY[-2::-1]:
        acc = acc * t + c
    return acc


def _box_t(x):
    s = jax.nn.sigmoid(x.T)
    z = s[:_DIM]
    Z = z + s[_DIM:] * (1.0 - z)
    return z, Z


def _tc_body(u_ref, c_ref, vols_ref, ints_ref, tv_ref, zZu_ref):
    j = pl.program_id(1)

    @pl.when(j == 0)
    def _():
        zu0, Zu0 = _box_t(u_ref[...])
        zZu_ref[:_DIM] = zu0
        zZu_ref[_DIM:] = Zu0
        tv_ref[0, 0, :] = jnp.sum(_f_poly(Zu0 - zu0), axis=0)

    zu = zZu_ref[:_DIM]
    Zu = zZu_ref[_DIM:]
    zc, Zc = _box_t(c_ref[...])
    t = jnp.concatenate(
        [Zc - zc, jnp.minimum(Zc, Zu) - jnp.maximum(zc, zu)], axis=0)
    f = _f_poly(t)
    vols_ref[0, 0, :] = jnp.sum(f[:_DIM], axis=0)
    ints_ref[0, 0, :] = jnp.sum(f[_DIM:], axis=0)


def _tc_compute(u_rows, ctx_rows, bb=512):
    nb = _BATCH // bb
    return pl.pallas_call(
        _tc_body,
        grid=(nb, _NPAIR),
        in_specs=[
            pl.BlockSpec((bb, 2 * _DIM), lambda i, j: (i, 0)),
            pl.BlockSpec((bb, 2 * _DIM), lambda i, j, nb=nb: (j * nb + i, 0)),
        ],
        out_specs=[
            pl.BlockSpec((1, 1, bb), lambda i, j: (j, 0, i)),
            pl.BlockSpec((1, 1, bb), lambda i, j: (j, 0, i)),
            pl.BlockSpec((1, 1, bb), lambda i, j: (0, 0, i)),
        ],
        out_shape=[
            jax.ShapeDtypeStruct((_NPAIR, 1, _BATCH), jnp.float32),
            jax.ShapeDtypeStruct((_NPAIR, 1, _BATCH), jnp.float32),
            jax.ShapeDtypeStruct((1, 1, _BATCH), jnp.float32),
        ],
        scratch_shapes=[pltpu.VMEM((2 * _DIM, bb), jnp.float32)],
    )(u_rows, ctx_rows)


def kernel(pos_u, pos_w, neg_w, W_word, W_ctx):
    pos_u = pos_u.astype(jnp.int32)
    idx_ctx = jnp.concatenate(
        [neg_w.astype(jnp.int32).T.reshape(-1), pos_w.astype(jnp.int32)])
    u_rows, ctx_rows = _sc_gather()(W_word, W_ctx, pos_u, idx_ctx)
    vols, ints, tv = _tc_compute(u_rows, ctx_rows)
    vols, ints, tv = vols[:, 0, :], ints[:, 0, :], tv[:, 0, :]
    return (tv[0], vols[_NNEG], vols[:_NNEG].T, ints[_NNEG], ints[:_NNEG].T)


# speedup vs baseline: 2.2823x; 1.3696x over previous
import functools

import jax
import jax.numpy as jnp
from jax import lax
from jax.experimental import pallas as pl
from jax.experimental.pallas import tpu as pltpu
from jax.experimental.pallas import tpu_sc as plsc

_DIM = 64
_BATCH = 16384
_NNEG = 20
_NPAIR = _NNEG + 1
_NW = 32
_CH = 128
_NB = 4

_U_PER_W = _BATCH // _NW
_C_PER_W = _NPAIR * _BATCH // _NW


def _gather_loop(table, idx_hbm, out_hbm, base, ngroups, idx_bufs, row_bufs,
                 isems, osems):
    def group(g, carry):
        gathers = []
        for b in range(_NB):
            off = base + (g * _NB + b) * _CH
            pltpu.sync_copy(idx_hbm.at[pl.ds(off, _CH)], idx_bufs[b])
            gathers.append(pltpu.async_copy(table.at[idx_bufs[b]], row_bufs[b],
                                            isems[b]))
        outs = []
        for b in range(_NB):
            gathers[b].wait()
            off = base + (g * _NB + b) * _CH
            outs.append(pltpu.async_copy(row_bufs[b],
                                         out_hbm.at[pl.ds(off, _CH)], osems[b]))
        for b in range(_NB):
            outs[b].wait()
        return carry

    lax.fori_loop(0, ngroups, group, 0)


def _sc_gather_body(w_word, w_ctx, idx_u, idx_c, out_u, out_c, *scratch):
    idx_bufs = scratch[0:_NB]
    row_bufs = scratch[_NB:2 * _NB]
    isems = scratch[2 * _NB:3 * _NB]
    osems = scratch[3 * _NB:4 * _NB]
    wid = lax.axis_index("s") * 2 + lax.axis_index("c")
    _gather_loop(w_word, idx_u, out_u, wid * _U_PER_W, _U_PER_W // (_NB * _CH),
                 idx_bufs, row_bufs, isems, osems)
    _gather_loop(w_ctx, idx_c, out_c, wid * _C_PER_W, _C_PER_W // (_NB * _CH),
                 idx_bufs, row_bufs, isems, osems)


@functools.cache
def _sc_gather():
    return pl.kernel(
        _sc_gather_body,
        out_type=(
            jax.ShapeDtypeStruct((_BATCH, 2 * _DIM), jnp.float32),
            jax.ShapeDtypeStruct((_NPAIR * _BATCH, 2 * _DIM), jnp.float32),
        ),
        mesh=plsc.VectorSubcoreMesh(core_axis_name="c", subcore_axis_name="s"),
        scratch_types=(
            [pltpu.VMEM((_CH,), jnp.int32) for _ in range(_NB)]
            + [pltpu.VMEM((_CH, 2 * _DIM), jnp.float32) for _ in range(_NB)]
            + [pltpu.SemaphoreType.DMA for _ in range(2 * _NB)]
        ),
    )


_POLY = (-0.3665129829491377, 0.7213459840780112, -0.07983222595229246,
         -0.004957223416339075, 0.0023628927052507724, 0.00022657838744351012,
         -0.00011899139943125192)


def _f_poly(t):
    acc = jnp.full_like(t, _POLY[-1])
    for c in _POLY[-2::-1]:
        acc = acc * t + c
    return acc


def _box_t(x):
    s = jax.nn.sigmoid(x.T)
    z = s[:_DIM]
    Z = z + s[_DIM:] * (1.0 - z)
    return z, Z


def _tc_body(u_ref, c_ref, vols_ref, ints_ref, tv_ref, zZu_ref):
    j = pl.program_id(1)

    @pl.when(j == 0)
    def _():
        zu0, Zu0 = _box_t(u_ref[...])
        zZu_ref[:_DIM] = zu0
        zZu_ref[_DIM:] = Zu0
        tv_ref[0, 0, :] = jnp.sum(_f_poly(Zu0 - zu0), axis=0)

    zu = zZu_ref[:_DIM]
    Zu = zZu_ref[_DIM:]
    zc, Zc = _box_t(c_ref[...])
    t = jnp.concatenate(
        [Zc - zc, jnp.minimum(Zc, Zu) - jnp.maximum(zc, zu)], axis=0)
    f = _f_poly(t)
    vols_ref[0, 0, :] = jnp.sum(f[:_DIM], axis=0)
    ints_ref[0, 0, :] = jnp.sum(f[_DIM:], axis=0)


def _tc_compute(u_rows, ctx_rows, bb=1024):
    nb = _BATCH // bb
    return pl.pallas_call(
        _tc_body,
        grid=(nb, _NPAIR),
        in_specs=[
            pl.BlockSpec((bb, 2 * _DIM), lambda i, j: (i, 0)),
            pl.BlockSpec((bb, 2 * _DIM), lambda i, j, nb=nb: (j * nb + i, 0)),
        ],
        out_specs=[
            pl.BlockSpec((1, 1, bb), lambda i, j: (j, 0, i)),
            pl.BlockSpec((1, 1, bb), lambda i, j: (j, 0, i)),
            pl.BlockSpec((1, 1, bb), lambda i, j: (0, 0, i)),
        ],
        out_shape=[
            jax.ShapeDtypeStruct((_NPAIR, 1, _BATCH), jnp.float32),
            jax.ShapeDtypeStruct((_NPAIR, 1, _BATCH), jnp.float32),
            jax.ShapeDtypeStruct((1, 1, _BATCH), jnp.float32),
        ],
        scratch_shapes=[pltpu.VMEM((2 * _DIM, bb), jnp.float32)],
    )(u_rows, ctx_rows)


def kernel(pos_u, pos_w, neg_w, W_word, W_ctx):
    pos_u = pos_u.astype(jnp.int32)
    idx_ctx = jnp.concatenate(
        [neg_w.astype(jnp.int32).T.reshape(-1), pos_w.astype(jnp.int32)])
    u_rows, ctx_rows = _sc_gather()(W_word, W_ctx, pos_u, idx_ctx)
    vols, ints, tv = _tc_compute(u_rows, ctx_rows)
    vols, ints, tv = vols[:, 0, :], ints[:, 0, :], tv[:, 0, :]
    return (tv[0], vols[_NNEG], vols[:_NNEG].T, ints[_NNEG], ints[:_NNEG].T)


# speedup vs baseline: 2.4631x; 1.0792x over previous
import functools

import jax
import jax.numpy as jnp
from jax import lax
from jax.experimental import pallas as pl
from jax.experimental.pallas import tpu as pltpu
from jax.experimental.pallas import tpu_sc as plsc

_DIM = 64
_BATCH = 16384
_NNEG = 20
_NPAIR = _NNEG + 1
_NW = 32
_CH = 128
_NB = 6
_NSLICE = 2
_BS = _BATCH // _NSLICE


def _gather_loop(table, idx_hbm, out_hbm, base, ngroups, nb, idx_bufs,
                 row_bufs, isems, osems):
    def group(g, carry):
        gathers = []
        for b in range(nb):
            off = base + (g * nb + b) * _CH
            pltpu.sync_copy(idx_hbm.at[pl.ds(off, _CH)], idx_bufs[b])
            gathers.append(pltpu.async_copy(table.at[idx_bufs[b]], row_bufs[b],
                                            isems[b]))
        outs = []
        for b in range(nb):
            gathers[b].wait()
            off = base + (g * nb + b) * _CH
            outs.append(pltpu.async_copy(row_bufs[b],
                                         out_hbm.at[pl.ds(off, _CH)], osems[b]))
        for b in range(nb):
            outs[b].wait()
        return carry

    lax.fori_loop(0, ngroups, group, 0)


def _sc_gather_body(w_word, w_ctx, idx_u, idx_c, out_u, out_c, *scratch):
    idx_bufs = scratch[0:_NB]
    row_bufs = scratch[_NB:2 * _NB]
    isems = scratch[2 * _NB:3 * _NB]
    osems = scratch[3 * _NB:4 * _NB]
    wid = lax.axis_index("s") * 2 + lax.axis_index("c")
    u_per_w = _BS // _NW
    c_per_w = _NPAIR * _BS // _NW
    _gather_loop(w_word, idx_u, out_u, wid * u_per_w, 1, 2,
                 idx_bufs, row_bufs, isems, osems)
    _gather_loop(w_ctx, idx_c, out_c, wid * c_per_w, c_per_w // (_CH * _NB),
                 _NB, idx_bufs, row_bufs, isems, osems)


@functools.cache
def _sc_gather():
    return pl.kernel(
        _sc_gather_body,
        out_type=(
            jax.ShapeDtypeStruct((_BS, 2 * _DIM), jnp.float32),
            jax.ShapeDtypeStruct((_NPAIR * _BS, 2 * _DIM), jnp.float32),
        ),
        mesh=plsc.VectorSubcoreMesh(core_axis_name="c", subcore_axis_name="s"),
        scratch_types=(
            [pltpu.VMEM((_CH,), jnp.int32) for _ in range(_NB)]
            + [pltpu.VMEM((_CH, 2 * _DIM), jnp.float32) for _ in range(_NB)]
            + [pltpu.SemaphoreType.DMA for _ in range(2 * _NB)]
        ),
    )


_POLY = (-0.3665129829491377, 0.7213459840780112, -0.07983222595229246,
         -0.004957223416339075, 0.0023628927052507724, 0.00022657838744351012,
         -0.00011899139943125192)


def _f_poly(t):
    acc = jnp.full_like(t, _POLY[-1])
    for c in _POLY[-2::-1]:
        acc = acc * t + c
    return acc


def _box_t(x):
    s = 1.0 / (1.0 + jnp.exp2(x.T * -1.4426950408889634))
    z = s[:_DIM]
    Z = z + s[_DIM:] * (1.0 - z)
    return z, Z


def _tc_body(u_ref, c_ref, vols_ref, ints_ref, tv_ref, zZu_ref):
    j = pl.program_id(1)

    @pl.when(j == 0)
    def _():
        zu0, Zu0 = _box_t(u_ref[...])
        zZu_ref[:_DIM] = zu0
        zZu_ref[_DIM:] = Zu0
        tv_ref[0, 0, :] = jnp.sum(_f_poly(Zu0 - zu0), axis=0)

    zu = zZu_ref[:_DIM]
    Zu = zZu_ref[_DIM:]
    zc, Zc = _box_t(c_ref[...])
    t = jnp.concatenate(
        [Zc - zc, jnp.minimum(Zc, Zu) - jnp.maximum(zc, zu)], axis=0)
    f = _f_poly(t)
    vols_ref[0, 0, :] = jnp.sum(f[:_DIM], axis=0)
    ints_ref[0, 0, :] = jnp.sum(f[_DIM:], axis=0)


def _tc_compute(u_rows, ctx_rows, bb=1024):
    nb = _BS // bb
    return pl.pallas_call(
        _tc_body,
        grid=(nb, _NPAIR),
        in_specs=[
            pl.BlockSpec((bb, 2 * _DIM), lambda i, j: (i, 0)),
            pl.BlockSpec((bb, 2 * _DIM), lambda i, j, nb=nb: (j * nb + i, 0)),
        ],
        out_specs=[
            pl.BlockSpec((1, 1, bb), lambda i, j: (j, 0, i)),
            pl.BlockSpec((1, 1, bb), lambda i, j: (j, 0, i)),
            pl.BlockSpec((1, 1, bb), lambda i, j: (0, 0, i)),
        ],
        out_shape=[
            jax.ShapeDtypeStruct((_NPAIR, 1, _BS), jnp.float32),
            jax.ShapeDtypeStruct((_NPAIR, 1, _BS), jnp.float32),
            jax.ShapeDtypeStruct((1, 1, _BS), jnp.float32),
        ],
        scratch_shapes=[pltpu.VMEM((2 * _DIM, bb), jnp.float32)],
    )(u_rows, ctx_rows)


def kernel(pos_u, pos_w, neg_w, W_word, W_ctx):
    pos_u = pos_u.astype(jnp.int32)
    pos_w = pos_w.astype(jnp.int32)
    neg_w = neg_w.astype(jnp.int32)
    vols_l, ints_l, tv_l = [], [], []
    for k in range(_NSLICE):
        sl = slice(k * _BS, (k + 1) * _BS)
        idx_ctx = jnp.concatenate([neg_w[sl].T.reshape(-1), pos_w[sl]])
        u_rows, ctx_rows = _sc_gather()(W_word, W_ctx, pos_u[sl], idx_ctx)
        vols, ints, tv = _tc_compute(u_rows, ctx_rows)
        vols_l.append(vols[:, 0, :])
        ints_l.append(ints[:, 0, :])
        tv_l.append(tv[0, 0, :])
    vols = jnp.concatenate(vols_l, axis=1)
    ints = jnp.concatenate(ints_l, axis=1)
    tv = jnp.concatenate(tv_l)
    return (tv, vols[_NNEG], vols[:_NNEG].T, ints[_NNEG], ints[:_NNEG].T)
